# Initial kernel scaffold; baseline (speedup 1.0000x reference)
import jax
import jax.numpy as jnp
from jax.experimental import pallas as pl


def kernel(logits, prefix_sum):
    raise NotImplementedError("write your pallas kernel here")


# speedup vs baseline: 5.7393x; 5.7393x over previous
import functools

import jax
import jax.numpy as jnp
from jax import lax
from jax.experimental import pallas as pl
from jax.experimental.pallas import tpu as pltpu
from jax.experimental.pallas import tpu_sc as plsc

N_TOK = 32768
NSEG = 16
NT = 16
CHUNK = N_TOK // NT
NV = CHUNK // 16
LN2 = 0.6931471805599453


def _mesh():
    return plsc.VectorSubcoreMesh(
        core_axis_name="c", subcore_axis_name="s", num_cores=1)


@functools.partial(
    pl.kernel,
    mesh=_mesh(),
    compiler_params=pltpu.CompilerParams(needs_layout_passes=False),
    out_type=jax.ShapeDtypeStruct((N_TOK,), jnp.float32),
    scratch_types=[
        pltpu.VMEM((CHUNK,), jnp.float32),
        pltpu.VMEM((CHUNK,), jnp.float32),
        pltpu.VMEM((NSEG,), jnp.int32),
        pltpu.VMEM((NSEG * 16,), jnp.float32),
        pltpu.VMEM((NT, 16), jnp.float32),
        pltpu.VMEM((NT, NSEG * 16), jnp.float32),
        pltpu.VMEM((16,), jnp.float32),
        pltpu.VMEM_SHARED((NT, 16), jnp.float32),
        pltpu.VMEM_SHARED((NT, NSEG * 16), jnp.float32),
    ],
)
def _jls(logits_hbm, ps_hbm, out_hbm, x_v, out_v, ps_v, part_v, buf_v,
         all_v, vec_v, sh_max, sh_part):
    w = lax.axis_index("s")
    base = w * CHUNK
    lane = lax.iota(jnp.int32, 16)

    pltpu.sync_copy(logits_hbm.at[pl.ds(base, CHUNK)], x_v)
    pltpu.sync_copy(ps_hbm, ps_v)
    psv = ps_v[...]

    m = x_v[pl.ds(0, 16)]
    for k in range(1, NV):
        m = jnp.maximum(m, x_v[pl.ds(k * 16, 16)])
    vec_v[...] = m
    pltpu.sync_copy(vec_v, sh_max.at[w])
    plsc.subcore_barrier()
    pltpu.sync_copy(sh_max, buf_v)
    g = buf_v[0]
    for t in range(1, NT):
        g = jnp.maximum(g, buf_v[t])
    gmax = jnp.max(g)

    for j in range(NSEG):
        s0 = 0 if j == 0 else psv[j - 1]
        s1 = psv[j]
        a = jnp.clip(s0 - base, 0, CHUNK)
        b = jnp.clip(s1 - base, 0, CHUNK)

        def body(k, acc, a=a, b=b):
            v = x_v[pl.ds(k * 16, 16)]
            idx = k * 16 + lane
            msk = (idx >= a) & (idx < b)
            return acc + jnp.where(msk, jnp.exp(v - gmax), 0.0)

        acc = lax.fori_loop(a // 16, (b + 15) // 16, body,
                            jnp.zeros((16,), jnp.float32))
        part_v[pl.ds(j * 16, 16)] = acc
    pltpu.sync_copy(part_v, sh_part.at[w])
    plsc.subcore_barrier()
    pltpu.sync_copy(sh_part, all_v)

    accs = [all_v[0, pl.ds(j * 16, 16)] for j in range(NSEG)]
    for t in range(1, NT):
        for j in range(NSEG):
            accs[j] = accs[j] + all_v[t, pl.ds(j * 16, 16)]
    s_vec = jnp.zeros((16,), jnp.float32)
    for j in range(NSEG):
        s_vec = jnp.where(lane == j, jnp.sum(accs[j]), s_vec)

    bits = plsc.bitcast(s_vec, jnp.int32)
    e = (bits >> 23) - 127
    mant = plsc.bitcast((bits & 0x7FFFFF) | 0x3F800000, jnp.float32)
    big = mant > 1.4142135623730951
    mant = jnp.where(big, mant * 0.5, mant)
    e = jnp.where(big, e + 1, e)
    r = (mant - 1.0) / (mant + 1.0)
    r2 = r * r
    poly = r * (2.0 + r2 * (2.0 / 3.0 + r2 * (2.0 / 5.0
               + r2 * (2.0 / 7.0 + r2 * (2.0 / 9.0)))))
    adj = e.astype(jnp.float32) * LN2 + poly + gmax

    for j in range(NSEG):
        s0 = 0 if j == 0 else psv[j - 1]
        s1 = psv[j]
        a = jnp.clip(s0 - base, 0, CHUNK)
        b = jnp.clip(s1 - base, 0, CHUNK)
        aj = adj[j]

        def body3(k, carry, a=a, b=b, aj=aj):
            sl = pl.ds(k * 16, 16)
            idx = k * 16 + lane
            msk = (idx >= a) & (idx < b)
            out_v[sl] = jnp.where(msk, x_v[sl] - aj, out_v[sl])
            return carry

        lax.fori_loop(a // 16, (b + 15) // 16, body3, 0)

    pltpu.sync_copy(out_v, out_hbm.at[pl.ds(base, CHUNK)])


def kernel(logits, prefix_sum):
    return _jls(logits, prefix_sum)


# speedup vs baseline: 6.1197x; 1.0663x over previous
import functools

import jax
import jax.numpy as jnp
from jax import lax
from jax.experimental import pallas as pl
from jax.experimental.pallas import tpu as pltpu
from jax.experimental.pallas import tpu_sc as plsc

N_TOK = 32768
NSEG = 16
NT = 16
CHUNK = N_TOK // NT
NV = CHUNK // 16
LN2 = 0.6931471805599453


def _mesh():
    return plsc.VectorSubcoreMesh(
        core_axis_name="c", subcore_axis_name="s", num_cores=1)


@functools.partial(
    pl.kernel,
    mesh=_mesh(),
    compiler_params=pltpu.CompilerParams(needs_layout_passes=False),
    out_type=jax.ShapeDtypeStruct((N_TOK,), jnp.float32),
    scratch_types=[
        pltpu.VMEM((CHUNK,), jnp.float32),
        pltpu.VMEM((CHUNK,), jnp.float32),
        pltpu.VMEM((NSEG,), jnp.int32),
        pltpu.VMEM(((NV + 1) * 16,), jnp.float32),
        pltpu.VMEM((NSEG * 16,), jnp.float32),
        pltpu.VMEM((NT, 16), jnp.float32),
        pltpu.VMEM((NT, NSEG * 16), jnp.float32),
        pltpu.VMEM((16,), jnp.float32),
        pltpu.VMEM_SHARED((NT, 16), jnp.float32),
        pltpu.VMEM_SHARED((NT, NSEG * 16), jnp.float32),
    ],
)
def _jls(logits_hbm, ps_hbm, out_hbm, x_v, out_v, ps_v, p_v, part_v,
         buf_v, all_v, vec_v, sh_max, sh_part):
    w = lax.axis_index("s")
    base = w * CHUNK
    lane = lax.iota(jnp.int32, 16)

    pltpu.sync_copy(logits_hbm.at[pl.ds(base, CHUNK)], x_v)
    pltpu.sync_copy(ps_hbm, ps_v)
    psv = ps_v[...]

    m0 = x_v[pl.ds(0, 16)]
    m1 = x_v[pl.ds(16, 16)]
    m2 = x_v[pl.ds(32, 16)]
    m3 = x_v[pl.ds(48, 16)]
    for k in range(4, NV, 4):
        m0 = jnp.maximum(m0, x_v[pl.ds(k * 16, 16)])
        m1 = jnp.maximum(m1, x_v[pl.ds((k + 1) * 16, 16)])
        m2 = jnp.maximum(m2, x_v[pl.ds((k + 2) * 16, 16)])
        m3 = jnp.maximum(m3, x_v[pl.ds((k + 3) * 16, 16)])
    vec_v[...] = jnp.maximum(jnp.maximum(m0, m1), jnp.maximum(m2, m3))
    pltpu.sync_copy(vec_v, sh_max.at[w])
    plsc.subcore_barrier()
    pltpu.sync_copy(sh_max, buf_v)
    g = buf_v[0]
    for t in range(1, NT):
        g = jnp.maximum(g, buf_v[t])
    gmax = jnp.max(g)

    pacc = jnp.zeros((16,), jnp.float32)
    p_v[pl.ds(0, 16)] = pacc
    for k in range(NV):
        pacc = pacc + jnp.exp(x_v[pl.ds(k * 16, 16)] - gmax)
        p_v[pl.ds((k + 1) * 16, 16)] = pacc

    for j in range(NSEG):
        s0 = 0 if j == 0 else psv[j - 1]
        s1 = psv[j]
        a = jnp.clip(s0 - base, 0, CHUNK)
        b = jnp.clip(s1 - base, 0, CHUNK)
        ka = (a + 15) // 16
        kb = b // 16
        sel = ka <= kb
        interior = jnp.where(
            sel, p_v[pl.ds(kb * 16, 16)] - p_v[pl.ds(ka * 16, 16)], 0.0)
        kl = jnp.minimum(a // 16, NV - 1)
        idxl = kl * 16 + lane
        vl = jnp.exp(x_v[pl.ds(kl * 16, 16)] - gmax)
        left = jnp.where((idxl >= a) & (idxl < b) & (idxl < ka * 16),
                         vl, 0.0)
        kr = jnp.minimum(kb, NV - 1)
        idxr = kr * 16 + lane
        vr = jnp.exp(x_v[pl.ds(kr * 16, 16)] - gmax)
        right = jnp.where((idxr >= a) & (idxr < b) & (idxr >= kb * 16)
                          & sel, vr, 0.0)
        part_v[pl.ds(j * 16, 16)] = interior + left + right

    pltpu.sync_copy(part_v, sh_part.at[w])
    plsc.subcore_barrier()
    pltpu.sync_copy(sh_part, all_v)
    accs = [all_v[0, pl.ds(j * 16, 16)] for j in range(NSEG)]
    for t in range(1, NT):
        for j in range(NSEG):
            accs[j] = accs[j] + all_v[t, pl.ds(j * 16, 16)]
    s_vec = jnp.zeros((16,), jnp.float32)
    for j in range(NSEG):
        s_vec = jnp.where(lane == j, jnp.sum(accs[j]), s_vec)

    bits = plsc.bitcast(s_vec, jnp.int32)
    e = (bits >> 23) - 127
    mant = plsc.bitcast((bits & 0x7FFFFF) | 0x3F800000, jnp.float32)
    big = mant > 1.4142135623730951
    mant = jnp.where(big, mant * 0.5, mant)
    e = jnp.where(big, e + 1, e)
    r = (mant - 1.0) / (mant + 1.0)
    r2 = r * r
    poly = r * (2.0 + r2 * (2.0 / 3.0 + r2 * (2.0 / 5.0
               + r2 * (2.0 / 7.0 + r2 * (2.0 / 9.0)))))
    adj = e.astype(jnp.float32) * LN2 + poly + gmax

    for j in range(NSEG):
        s0 = 0 if j == 0 else psv[j - 1]
        s1 = psv[j]
        a = jnp.clip(s0 - base, 0, CHUNK)
        b = jnp.clip(s1 - base, 0, CHUNK)
        ka = (a + 15) // 16
        kb = b // 16
        sel = ka <= kb
        aj = adj[j]

        def body3(k, carry, aj=aj):
            sl = pl.ds(k * 16, 16)
            out_v[sl] = x_v[sl] - aj
            return carry

        lax.fori_loop(ka, kb, body3, 0)

        kl = jnp.minimum(a // 16, NV - 1)
        sll = pl.ds(kl * 16, 16)
        idxl = kl * 16 + lane
        ml = (idxl >= a) & (idxl < b) & (idxl < ka * 16)
        out_v[sll] = jnp.where(ml, x_v[sll] - aj, out_v[sll])
        kr = jnp.minimum(kb, NV - 1)
        slr = pl.ds(kr * 16, 16)
        idxr = kr * 16 + lane
        mr = (idxr >= a) & (idxr < b) & (idxr >= kb * 16) & sel
        out_v[slr] = jnp.where(mr, x_v[slr] - aj, out_v[slr])

    pltpu.sync_copy(out_v, out_hbm.at[pl.ds(base, CHUNK)])


def kernel(logits, prefix_sum):
    return _jls(logits, prefix_sum)


# speedup vs baseline: 6.4738x; 1.0579x over previous
import functools

import jax
import jax.numpy as jnp
from jax import lax
from jax.experimental import pallas as pl
from jax.experimental.pallas import tpu as pltpu
from jax.experimental.pallas import tpu_sc as plsc

N_TOK = 32768
NSEG = 16
NT = 16
CHUNK = N_TOK // NT
NV = CHUNK // 16
LN2 = 0.6931471805599453


def _mesh():
    return plsc.VectorSubcoreMesh(
        core_axis_name="c", subcore_axis_name="s", num_cores=1)


@functools.partial(
    pl.kernel,
    mesh=_mesh(),
    compiler_params=pltpu.CompilerParams(needs_layout_passes=False),
    out_type=jax.ShapeDtypeStruct((N_TOK,), jnp.float32),
    scratch_types=[
        pltpu.VMEM((CHUNK,), jnp.float32),
        pltpu.VMEM((CHUNK,), jnp.float32),
        pltpu.VMEM((NSEG,), jnp.int32),
        pltpu.VMEM(((NV + 1) * 16,), jnp.float32),
        pltpu.VMEM((NT, 128), jnp.float32),
        pltpu.VMEM((NT, 128), jnp.float32),
        pltpu.VMEM((128,), jnp.float32),
        pltpu.VMEM_SHARED((2 * NT, 128), jnp.float32),
    ],
)
def _jls(logits_hbm, ps_hbm, out_hbm, x_v, out_v, ps_v, p_v,
         buf_v, buf2_v, vec_v, sh):
    w = lax.axis_index("s")
    base = w * CHUNK
    lane = lax.iota(jnp.int32, 16)

    pltpu.sync_copy(logits_hbm.at[pl.ds(base, CHUNK)], x_v)
    pltpu.sync_copy(ps_hbm, ps_v)
    psv = ps_v[...]

    m0 = x_v[pl.ds(0, 16)]
    m1 = x_v[pl.ds(16, 16)]
    m2 = x_v[pl.ds(32, 16)]
    m3 = x_v[pl.ds(48, 16)]
    for k in range(4, NV, 4):
        m0 = jnp.maximum(m0, x_v[pl.ds(k * 16, 16)])
        m1 = jnp.maximum(m1, x_v[pl.ds((k + 1) * 16, 16)])
        m2 = jnp.maximum(m2, x_v[pl.ds((k + 2) * 16, 16)])
        m3 = jnp.maximum(m3, x_v[pl.ds((k + 3) * 16, 16)])
    vec_v[pl.ds(0, 16)] = jnp.maximum(jnp.maximum(m0, m1),
                                      jnp.maximum(m2, m3))
    pltpu.sync_copy(vec_v, sh.at[w])
    plsc.subcore_barrier()
    pltpu.sync_copy(sh.at[pl.ds(0, NT)], buf_v)
    g = buf_v[0, pl.ds(0, 16)]
    for t in range(1, NT):
        g = jnp.maximum(g, buf_v[t, pl.ds(0, 16)])
    gmax = jnp.max(g)

    pacc = jnp.zeros((16,), jnp.float32)
    p_v[pl.ds(0, 16)] = pacc
    for k in range(NV):
        pacc = pacc + jnp.exp(x_v[pl.ds(k * 16, 16)] - gmax)
        p_v[pl.ds((k + 1) * 16, 16)] = pacc

    pub = jnp.zeros((16,), jnp.float32)
    for j in range(NSEG):
        s0 = 0 if j == 0 else psv[j - 1]
        s1 = psv[j]
        a = jnp.clip(s0 - base, 0, CHUNK)
        b = jnp.clip(s1 - base, 0, CHUNK)
        ka = (a + 15) // 16
        kb = b // 16
        sel = ka <= kb
        interior = jnp.where(
            sel, p_v[pl.ds(kb * 16, 16)] - p_v[pl.ds(ka * 16, 16)], 0.0)
        kl = jnp.minimum(a // 16, NV - 1)
        idxl = kl * 16 + lane
        vl = jnp.exp(x_v[pl.ds(kl * 16, 16)] - gmax)
        left = jnp.where((idxl >= a) & (idxl < b) & (idxl < ka * 16),
                         vl, 0.0)
        kr = jnp.minimum(kb, NV - 1)
        idxr = kr * 16 + lane
        vr = jnp.exp(x_v[pl.ds(kr * 16, 16)] - gmax)
        right = jnp.where((idxr >= a) & (idxr < b) & (idxr >= kb * 16)
                          & sel, vr, 0.0)
        pub = jnp.where(lane == j, jnp.sum(interior + left + right), pub)

    vec_v[pl.ds(0, 16)] = pub
    pltpu.sync_copy(vec_v, sh.at[NT + w])
    plsc.subcore_barrier()
    pltpu.sync_copy(sh.at[pl.ds(NT, NT)], buf2_v)
    s_vec = buf2_v[0, pl.ds(0, 16)]
    for t in range(1, NT):
        s_vec = s_vec + buf2_v[t, pl.ds(0, 16)]

    bits = plsc.bitcast(s_vec, jnp.int32)
    e = (bits >> 23) - 127
    mant = plsc.bitcast((bits & 0x7FFFFF) | 0x3F800000, jnp.float32)
    big = mant > 1.4142135623730951
    mant = jnp.where(big, mant * 0.5, mant)
    e = jnp.where(big, e + 1, e)
    r = (mant - 1.0) / (mant + 1.0)
    r2 = r * r
    poly = r * (2.0 + r2 * (2.0 / 3.0 + r2 * (2.0 / 5.0
               + r2 * (2.0 / 7.0 + r2 * (2.0 / 9.0)))))
    adj = e.astype(jnp.float32) * LN2 + poly + gmax

    for j in range(NSEG):
        s0 = 0 if j == 0 else psv[j - 1]
        s1 = psv[j]
        a = jnp.clip(s0 - base, 0, CHUNK)
        b = jnp.clip(s1 - base, 0, CHUNK)
        ka = (a + 15) // 16
        kb = b // 16
        sel = ka <= kb
        aj = adj[j]

        def body3(k, carry, aj=aj):
            sl = pl.ds(k * 16, 16)
            out_v[sl] = x_v[sl] - aj
            return carry

        lax.fori_loop(ka, kb, body3, 0)

        kl = jnp.minimum(a // 16, NV - 1)
        sll = pl.ds(kl * 16, 16)
        idxl = kl * 16 + lane
        ml = (idxl >= a) & (idxl < b) & (idxl < ka * 16)
        out_v[sll] = jnp.where(ml, x_v[sll] - aj, out_v[sll])
        kr = jnp.minimum(kb, NV - 1)
        slr = pl.ds(kr * 16, 16)
        idxr = kr * 16 + lane
        mr = (idxr >= a) & (idxr < b) & (idxr >= kb * 16) & sel
        out_v[slr] = jnp.where(mr, x_v[slr] - aj, out_v[slr])

    pltpu.sync_copy(out_v, out_hbm.at[pl.ds(base, CHUNK)])


def kernel(logits, prefix_sum):
    return _jls(logits, prefix_sum)


# speedup vs baseline: 6.5796x; 1.0163x over previous
import functools

import jax
import jax.numpy as jnp
from jax import lax
from jax.experimental import pallas as pl
from jax.experimental.pallas import tpu as pltpu
from jax.experimental.pallas import tpu_sc as plsc

N_TOK = 32768
NSEG = 16
NT = 16
CHUNK = N_TOK // NT
NV = CHUNK // 16
LN2 = 0.6931471805599453


def _mesh():
    return plsc.VectorSubcoreMesh(
        core_axis_name="c", subcore_axis_name="s", num_cores=1)


@functools.partial(
    pl.kernel,
    mesh=_mesh(),
    compiler_params=pltpu.CompilerParams(needs_layout_passes=False),
    out_type=jax.ShapeDtypeStruct((N_TOK,), jnp.float32),
    scratch_types=[
        pltpu.VMEM((CHUNK,), jnp.float32),
        pltpu.VMEM((CHUNK,), jnp.float32),
        pltpu.VMEM((NSEG,), jnp.int32),
        pltpu.VMEM(((NV + 1) * 16,), jnp.float32),
        pltpu.VMEM((NT, 128), jnp.float32),
        pltpu.VMEM((128,), jnp.float32),
        pltpu.VMEM_SHARED((NT, 128), jnp.float32),
    ],
)
def _jls(logits_hbm, ps_hbm, out_hbm, x_v, out_v, ps_v, p_v,
         buf_v, vec_v, sh):
    w = lax.axis_index("s")
    base = w * CHUNK
    lane = lax.iota(jnp.int32, 16)

    pltpu.sync_copy(logits_hbm.at[pl.ds(base, CHUNK)], x_v)
    pltpu.sync_copy(ps_hbm, ps_v)
    psv = ps_v[...]

    m0 = x_v[pl.ds(0, 16)]
    m1 = x_v[pl.ds(16, 16)]
    m2 = x_v[pl.ds(32, 16)]
    m3 = x_v[pl.ds(48, 16)]
    for k in range(4, NV, 4):
        m0 = jnp.maximum(m0, x_v[pl.ds(k * 16, 16)])
        m1 = jnp.maximum(m1, x_v[pl.ds((k + 1) * 16, 16)])
        m2 = jnp.maximum(m2, x_v[pl.ds((k + 2) * 16, 16)])
        m3 = jnp.maximum(m3, x_v[pl.ds((k + 3) * 16, 16)])
    mvec = jnp.maximum(jnp.maximum(m0, m1), jnp.maximum(m2, m3))
    gmax = jnp.max(mvec)

    pacc = jnp.zeros((16,), jnp.float32)
    p_v[pl.ds(0, 16)] = pacc
    for k in range(NV):
        pacc = pacc + jnp.exp(x_v[pl.ds(k * 16, 16)] - gmax)
        p_v[pl.ds((k + 1) * 16, 16)] = pacc

    pub = jnp.zeros((16,), jnp.float32)
    for j in range(NSEG):
        s0 = 0 if j == 0 else psv[j - 1]
        s1 = psv[j]
        a = jnp.clip(s0 - base, 0, CHUNK)
        b = jnp.clip(s1 - base, 0, CHUNK)
        ka = (a + 15) // 16
        kb = b // 16
        sel = ka <= kb
        interior = jnp.where(
            sel, p_v[pl.ds(kb * 16, 16)] - p_v[pl.ds(ka * 16, 16)], 0.0)
        kl = jnp.minimum(a // 16, NV - 1)
        idxl = kl * 16 + lane
        vl = jnp.exp(x_v[pl.ds(kl * 16, 16)] - gmax)
        left = jnp.where((idxl >= a) & (idxl < b) & (idxl < ka * 16),
                         vl, 0.0)
        kr = jnp.minimum(kb, NV - 1)
        idxr = kr * 16 + lane
        vr = jnp.exp(x_v[pl.ds(kr * 16, 16)] - gmax)
        right = jnp.where((idxr >= a) & (idxr < b) & (idxr >= kb * 16)
                          & sel, vr, 0.0)
        pub = jnp.where(lane == j, jnp.sum(interior + left + right), pub)

    vec_v[pl.ds(0, 16)] = jnp.full((16,), gmax, jnp.float32)
    vec_v[pl.ds(16, 16)] = pub
    pltpu.sync_copy(vec_v, sh.at[w])
    plsc.subcore_barrier()
    pltpu.sync_copy(sh, buf_v)
    gv = buf_v[0, pl.ds(0, 16)]
    for t in range(1, NT):
        gv = jnp.maximum(gv, buf_v[t, pl.ds(0, 16)])
    s_vec = jnp.zeros((16,), jnp.float32)
    for t in range(NT):
        f = jnp.exp(buf_v[t, pl.ds(0, 16)] - gv)
        s_vec = s_vec + buf_v[t, pl.ds(16, 16)] * f

    bits = plsc.bitcast(s_vec, jnp.int32)
    e = (bits >> 23) - 127
    mant = plsc.bitcast((bits & 0x7FFFFF) | 0x3F800000, jnp.float32)
    big = mant > 1.4142135623730951
    mant = jnp.where(big, mant * 0.5, mant)
    e = jnp.where(big, e + 1, e)
    r = (mant - 1.0) / (mant + 1.0)
    r2 = r * r
    poly = r * (2.0 + r2 * (2.0 / 3.0 + r2 * (2.0 / 5.0
               + r2 * (2.0 / 7.0 + r2 * (2.0 / 9.0)))))
    adj = e.astype(jnp.float32) * LN2 + poly + gv

    for j in range(NSEG):
        s0 = 0 if j == 0 else psv[j - 1]
        s1 = psv[j]
        a = jnp.clip(s0 - base, 0, CHUNK)
        b = jnp.clip(s1 - base, 0, CHUNK)
        ka = (a + 15) // 16
        kb = b // 16
        sel = ka <= kb
        aj = adj[j]

        def body3(k, carry, aj=aj):
            sl = pl.ds(k * 16, 16)
            out_v[sl] = x_v[sl] - aj
            return carry

        lax.fori_loop(ka, kb, body3, 0)

        kl = jnp.minimum(a // 16, NV - 1)
        sll = pl.ds(kl * 16, 16)
        idxl = kl * 16 + lane
        ml = (idxl >= a) & (idxl < b) & (idxl < ka * 16)
        out_v[sll] = jnp.where(ml, x_v[sll] - aj, out_v[sll])
        kr = jnp.minimum(kb, NV - 1)
        slr = pl.ds(kr * 16, 16)
        idxr = kr * 16 + lane
        mr = (idxr >= a) & (idxr < b) & (idxr >= kb * 16) & sel
        out_v[slr] = jnp.where(mr, x_v[slr] - aj, out_v[slr])

    pltpu.sync_copy(out_v, out_hbm.at[pl.ds(base, CHUNK)])


def kernel(logits, prefix_sum):
    return _jls(logits, prefix_sum)
